# initial kernel scaffold (unmeasured)
import jax
import jax.numpy as jnp
from jax import lax
from jax.experimental import pallas as pl
from jax.experimental.pallas import tpu as pltpu

N_DEV = 8
BLK = 64
STRIDE = 4


def _ring_allgather(kv):
    two, B, S, H, D = kv.shape

    def body(kv_ref, out_ref, send_sems, recv_sems):
        my = lax.axis_index("i")
        left = lax.rem(my - 1 + N_DEV, N_DEV)
        right = lax.rem(my + 1, N_DEV)

        barrier = pltpu.get_barrier_semaphore()
        for nbr in (left, right):
            pl.semaphore_signal(
                barrier, inc=1,
                device_id=(nbr,), device_id_type=pl.DeviceIdType.MESH,
            )
        pl.semaphore_wait(barrier, 2)

        out_ref[my] = kv_ref[...]

        for h in range(N_DEV - 1):
            o = lax.rem(my - h + N_DEV, N_DEV)
            rdma = pltpu.make_async_remote_copy(
                src_ref=out_ref.at[o],
                dst_ref=out_ref.at[o],
                send_sem=send_sems.at[h],
                recv_sem=recv_sems.at[h],
                device_id=(right,),
                device_id_type=pl.DeviceIdType.MESH,
            )
            rdma.start()
            rdma.wait()

    return pl.pallas_call(
        body,
        out_shape=jax.ShapeDtypeStruct((N_DEV, two, B, S, H, D), kv.dtype),
        in_specs=[pl.BlockSpec(memory_space=pltpu.VMEM)],
        out_specs=pl.BlockSpec(memory_space=pltpu.VMEM),
        scratch_shapes=[
            pltpu.SemaphoreType.DMA((N_DEV - 1,)),
            pltpu.SemaphoreType.DMA((N_DEV - 1,)),
        ],
        compiler_params=pltpu.CompilerParams(collective_id=0),
    )(kv)


def kernel(x, Wq, K_ext, V_ext, Wo):
    B, Sq, E = x.shape
    _, _, H, D = K_ext.shape
    S = K_ext.shape[1]

    gathered = _ring_allgather(jnp.stack([K_ext, V_ext]))

    def by_residue(t):
        t = t.reshape(N_DEV, B, S // (STRIDE * BLK), STRIDE, BLK, H, D)
        t = t.transpose(3, 1, 0, 2, 4, 5, 6)
        return t.reshape(STRIDE, B, N_DEV * S // STRIDE, H, D)

    Kr = by_residue(gathered[:, 0])
    Vr = by_residue(gathered[:, 1])

    Q = (x.reshape(B * Sq, E) @ Wq).reshape(B, Sq, H, D)
    Qr = Q.reshape(B, Sq // (STRIDE * BLK), STRIDE, BLK, H, D)
    Qr = Qr.transpose(2, 0, 1, 3, 4, 5).reshape(STRIDE, B, Sq // STRIDE, H, D)

    s = jnp.einsum("rbihd,rbjhd->rbhij", Qr, Kr) * 0.125
    s = s - s.max(axis=-1, keepdims=True)
    w = jnp.exp(s)
    w = w / w.sum(axis=-1, keepdims=True)
    ctx = jnp.einsum("rbhij,rbjhd->rbihd", w, Vr)

    ctx = ctx.reshape(STRIDE, B, Sq // (STRIDE * BLK), BLK, H, D)
    ctx = ctx.transpose(1, 2, 0, 3, 4, 5).reshape(B, Sq, H * D)
    return ctx @ Wo


# baseline (device time: 626731 ns/iter reference)
import jax
import jax.numpy as jnp
from jax import lax
from jax.experimental import pallas as pl
from jax.experimental.pallas import tpu as pltpu

N_DEV = 8
BLK = 64
STRIDE = 4


def _ring_allgather(kv):
    two, B, S, F = kv.shape

    def body(kv_ref, out_ref, send_sems, recv_sems):
        my = lax.axis_index("i")
        left = lax.rem(my - 1 + N_DEV, N_DEV)
        right = lax.rem(my + 1, N_DEV)

        barrier = pltpu.get_barrier_semaphore()
        for nbr in (left, right):
            pl.semaphore_signal(
                barrier, inc=1,
                device_id=(nbr,), device_id_type=pl.DeviceIdType.MESH,
            )
        pl.semaphore_wait(barrier, 2)

        out_ref[my] = kv_ref[...]

        for h in range(N_DEV - 1):
            o = lax.rem(my - h + N_DEV, N_DEV)
            rdma = pltpu.make_async_remote_copy(
                src_ref=out_ref.at[o],
                dst_ref=out_ref.at[o],
                send_sem=send_sems.at[h],
                recv_sem=recv_sems.at[h],
                device_id=(right,),
                device_id_type=pl.DeviceIdType.MESH,
            )
            rdma.start()
            rdma.wait()

    return pl.pallas_call(
        body,
        out_shape=jax.ShapeDtypeStruct((N_DEV, two, B, S, F), kv.dtype),
        in_specs=[pl.BlockSpec(memory_space=pltpu.VMEM)],
        out_specs=pl.BlockSpec(memory_space=pltpu.VMEM),
        scratch_shapes=[
            pltpu.SemaphoreType.DMA((N_DEV - 1,)),
            pltpu.SemaphoreType.DMA((N_DEV - 1,)),
        ],
        compiler_params=pltpu.CompilerParams(collective_id=0),
    )(kv)


def kernel(x, Wq, K_ext, V_ext, Wo):
    B, Sq, E = x.shape
    _, _, H, D = K_ext.shape
    S = K_ext.shape[1]

    kv = jnp.stack([K_ext, V_ext]).reshape(2, B, S, H * D)
    gathered = _ring_allgather(kv)
    gathered = gathered.reshape(N_DEV, 2, B, S, H, D)

    def by_residue(t):
        t = t.reshape(N_DEV, B, S // (STRIDE * BLK), STRIDE, BLK, H, D)
        t = t.transpose(3, 1, 0, 2, 4, 5, 6)
        return t.reshape(STRIDE, B, N_DEV * S // STRIDE, H, D)

    Kr = by_residue(gathered[:, 0])
    Vr = by_residue(gathered[:, 1])

    Q = (x.reshape(B * Sq, E) @ Wq).reshape(B, Sq, H, D)
    Qr = Q.reshape(B, Sq // (STRIDE * BLK), STRIDE, BLK, H, D)
    Qr = Qr.transpose(2, 0, 1, 3, 4, 5).reshape(STRIDE, B, Sq // STRIDE, H, D)

    s = jnp.einsum("rbihd,rbjhd->rbhij", Qr, Kr) * 0.125
    s = s - s.max(axis=-1, keepdims=True)
    w = jnp.exp(s)
    w = w / w.sum(axis=-1, keepdims=True)
    ctx = jnp.einsum("rbhij,rbjhd->rbihd", w, Vr)

    ctx = ctx.reshape(STRIDE, B, Sq // (STRIDE * BLK), BLK, H, D)
    ctx = ctx.transpose(1, 2, 0, 3, 4, 5).reshape(B, Sq, H * D)
    return ctx @ Wo


# device time: 468142 ns/iter; 1.3388x vs baseline; 1.3388x over previous
import jax
import jax.numpy as jnp
import numpy as np
from jax import lax
from jax.experimental import pallas as pl
from jax.experimental.pallas import tpu as pltpu

N_DEV = 8
BLK = 64
STRIDE = 4

PERM = [0, 1, 2, 3, 7, 6, 5, 4]
POS = [0, 1, 2, 3, 7, 6, 5, 4]
NEXT = [1, 2, 3, 7, 0, 4, 5, 6]
PREV = [4, 0, 1, 2, 5, 6, 7, 3]


def _lut(table, idx):
    out = jnp.int32(table[0])
    for k in range(1, N_DEV):
        out = jnp.where(idx == k, jnp.int32(table[k]), out)
    return out


def _bidir_allgather(kv_lo, kv_hi):
    two, B, S, F = kv_lo.shape

    def body(lo_ref, hi_ref, out_lo, out_hi,
             cw_send, cw_recv, ccw_send, ccw_recv):
        my = lax.axis_index("i")
        r = _lut(POS, my)
        right = _lut(NEXT, my)
        left = _lut(PREV, my)

        barrier = pltpu.get_barrier_semaphore()
        for nbr in (left, right):
            pl.semaphore_signal(
                barrier, inc=1,
                device_id=(nbr,), device_id_type=pl.DeviceIdType.MESH,
            )
        pl.semaphore_wait(barrier, 2)

        out_lo[r] = lo_ref[...]
        out_hi[r] = hi_ref[...]

        for h in range(N_DEV - 1):
            s_cw = lax.rem(r - h + N_DEV, N_DEV)
            s_ccw = lax.rem(r + h, N_DEV)
            rd_cw = pltpu.make_async_remote_copy(
                src_ref=out_lo.at[s_cw],
                dst_ref=out_lo.at[s_cw],
                send_sem=cw_send.at[h],
                recv_sem=cw_recv.at[h],
                device_id=(right,),
                device_id_type=pl.DeviceIdType.MESH,
            )
            rd_ccw = pltpu.make_async_remote_copy(
                src_ref=out_hi.at[s_ccw],
                dst_ref=out_hi.at[s_ccw],
                send_sem=ccw_send.at[h],
                recv_sem=ccw_recv.at[h],
                device_id=(left,),
                device_id_type=pl.DeviceIdType.MESH,
            )
            rd_cw.start()
            rd_ccw.start()
            rd_cw.wait()
            rd_ccw.wait()

    shp = jax.ShapeDtypeStruct((N_DEV, two, B, S, F), kv_lo.dtype)
    return pl.pallas_call(
        body,
        out_shape=(shp, shp),
        in_specs=[pl.BlockSpec(memory_space=pltpu.VMEM)] * 2,
        out_specs=(pl.BlockSpec(memory_space=pltpu.VMEM),) * 2,
        scratch_shapes=[
            pltpu.SemaphoreType.DMA((N_DEV - 1,)),
            pltpu.SemaphoreType.DMA((N_DEV - 1,)),
            pltpu.SemaphoreType.DMA((N_DEV - 1,)),
            pltpu.SemaphoreType.DMA((N_DEV - 1,)),
        ],
        compiler_params=pltpu.CompilerParams(collective_id=0),
    )(kv_lo, kv_hi)


def kernel(x, Wq, K_ext, V_ext, Wo):
    B, Sq, E = x.shape
    _, _, H, D = K_ext.shape
    S = K_ext.shape[1]
    Hh = H // 2

    lo = jnp.stack([K_ext[:, :, :Hh], V_ext[:, :, :Hh]]).reshape(2, B, S, Hh * D)
    hi = jnp.stack([K_ext[:, :, Hh:], V_ext[:, :, Hh:]]).reshape(2, B, S, Hh * D)
    g_lo, g_hi = _bidir_allgather(lo, hi)

    order = np.array(POS)
    g_lo = g_lo[order].reshape(N_DEV, 2, B, S, Hh, D)
    g_hi = g_hi[order].reshape(N_DEV, 2, B, S, Hh, D)
    gathered = jnp.concatenate([g_lo, g_hi], axis=4)

    def by_residue(t):
        t = t.reshape(N_DEV, B, S // (STRIDE * BLK), STRIDE, BLK, H, D)
        t = t.transpose(3, 1, 0, 2, 4, 5, 6)
        return t.reshape(STRIDE, B, N_DEV * S // STRIDE, H, D)

    Kr = by_residue(gathered[:, 0])
    Vr = by_residue(gathered[:, 1])

    Q = (x.reshape(B * Sq, E) @ Wq).reshape(B, Sq, H, D)
    Qr = Q.reshape(B, Sq // (STRIDE * BLK), STRIDE, BLK, H, D)
    Qr = Qr.transpose(2, 0, 1, 3, 4, 5).reshape(STRIDE, B, Sq // STRIDE, H, D)

    s = jnp.einsum("rbihd,rbjhd->rbhij", Qr, Kr) * 0.125
    s = s - s.max(axis=-1, keepdims=True)
    w = jnp.exp(s)
    w = w / w.sum(axis=-1, keepdims=True)
    ctx = jnp.einsum("rbhij,rbjhd->rbihd", w, Vr)

    ctx = ctx.reshape(STRIDE, B, Sq // (STRIDE * BLK), BLK, H, D)
    ctx = ctx.transpose(1, 2, 0, 3, 4, 5).reshape(B, Sq, H * D)
    return ctx @ Wo


# device time: 203492 ns/iter; 3.0799x vs baseline; 2.3005x over previous
import jax
import jax.numpy as jnp
from jax import lax
from jax.experimental import pallas as pl
from jax.experimental.pallas import tpu as pltpu

N_DEV = 8
BLK = 64
STRIDE = 4

POS = [0, 1, 2, 3, 7, 6, 5, 4]
NEXT = [1, 2, 3, 7, 0, 4, 5, 6]
PREV = [4, 0, 1, 2, 5, 6, 7, 3]


def _lut(table, idx):
    out = jnp.int32(table[0])
    for k in range(1, N_DEV):
        out = jnp.where(idx == k, jnp.int32(table[k]), out)
    return out


def kernel(x, Wq, K_ext, V_ext, Wo):
    B, Sq, E = x.shape
    _, S, H, D = K_ext.shape
    F = H * D
    Hh = H // 2
    Fh = Hh * D
    R = Sq // STRIDE
    NB = Sq // (STRIDE * BLK)

    def body(x_ref, wq_ref, k_ref, v_ref, wo_ref, o_ref,
             klo, vlo, khi, vhi, qbuf, acc, lsum, ctx,
             klo_s, klo_r, vlo_s, vlo_r, khi_s, khi_r, vhi_s, vhi_r):
        my = lax.axis_index("i")
        p = _lut(POS, my)
        right = _lut(NEXT, my)
        left = _lut(PREV, my)

        barrier = pltpu.get_barrier_semaphore()
        for nbr in (left, right):
            pl.semaphore_signal(
                barrier, inc=1,
                device_id=(nbr,), device_id_type=pl.DeviceIdType.MESH,
            )
        pl.semaphore_wait(barrier, 2)

        klo[p] = k_ref[:, :, :Fh]
        vlo[p] = v_ref[:, :, :Fh]
        khi[p] = k_ref[:, :, Fh:]
        vhi[p] = v_ref[:, :, Fh:]

        acc[...] = jnp.zeros_like(acc)
        lsum[...] = jnp.zeros_like(lsum)

        def hop_rdmas(h):
            s_cw = lax.rem(p - h + N_DEV, N_DEV)
            s_ccw = lax.rem(p + h, N_DEV)
            return [
                pltpu.make_async_remote_copy(
                    src_ref=buf.at[slot], dst_ref=buf.at[slot],
                    send_sem=sems.at[h], recv_sem=rsems.at[h],
                    device_id=(tgt,), device_id_type=pl.DeviceIdType.MESH,
                )
                for buf, sems, rsems, tgt, slot in (
                    (klo, klo_s, klo_r, right, s_cw),
                    (vlo, vlo_s, vlo_r, right, s_cw),
                    (khi, khi_s, khi_r, left, s_ccw),
                    (vhi, vhi_s, vhi_r, left, s_ccw),
                )
            ]

        def process_pair(h):
            s_cw = lax.rem(p - h + N_DEV, N_DEV)
            s_ccw = lax.rem(p + h, N_DEV)

            def rb_body(i, _):
                r = i // B
                b = lax.rem(i, B)
                for half, kbuf, vbuf, slot in (
                    (0, klo, vlo, s_cw),
                    (1, khi, vhi, s_ccw),
                ):
                    kc = jnp.concatenate(
                        [kbuf[slot, b, pl.ds((r + STRIDE * a) * BLK, BLK), :]
                         for a in range(NB)], axis=0)
                    vc = jnp.concatenate(
                        [vbuf[slot, b, pl.ds((r + STRIDE * a) * BLK, BLK), :]
                         for a in range(NB)], axis=0)
                    q = qbuf[r, b, :, pl.ds(half * Fh, Fh)]
                    s = jnp.einsum(
                        "ihd,jhd->hij",
                        q.reshape(R, Hh, D),
                        kc.reshape(NB * BLK, Hh, D),
                        preferred_element_type=jnp.float32,
                    ) * 0.125
                    e = jnp.exp(s)
                    lsum[r, b, pl.ds(half * Hh, Hh), :] = (
                        lsum[r, b, pl.ds(half * Hh, Hh), :] + e.sum(axis=-1)
                    )
                    c = jnp.einsum(
                        "hij,jhd->ihd", e, vc.reshape(NB * BLK, Hh, D),
                        preferred_element_type=jnp.float32,
                    ).reshape(R, Fh)
                    acc[r, b, :, pl.ds(half * Fh, Fh)] = (
                        acc[r, b, :, pl.ds(half * Fh, Fh)] + c
                    )
                return 0

            lax.fori_loop(0, STRIDE * B, rb_body, 0)

        rdmas0 = hop_rdmas(0)
        for rdma in rdmas0:
            rdma.start()

        for r in range(STRIDE):
            for b in range(B):
                xs = jnp.concatenate(
                    [x_ref[b, (r + STRIDE * a) * BLK:
                           (r + STRIDE * a + 1) * BLK, :] for a in range(NB)],
                    axis=0,
                )
                qbuf[r, b] = jnp.dot(
                    xs, wq_ref[...], preferred_element_type=jnp.float32
                )

        process_pair(0)
        for rdma in rdmas0:
            rdma.wait()

        def hop_body(h, _):
            rdmas = hop_rdmas(h)
            for rdma in rdmas:
                rdma.start()
            process_pair(h)
            for rdma in rdmas:
                rdma.wait()
            return 0

        lax.fori_loop(1, N_DEV - 1, hop_body, 0)

        process_pair(N_DEV - 1)

        for r in range(STRIDE):
            a_r = acc[r]
            l_r = lsum[r]
            ctxr = a_r.reshape(B, R, H, D) / jnp.transpose(
                l_r, (0, 2, 1)
            )[:, :, :, None]
            ctxr = ctxr.reshape(B, R, F)
            for a in range(NB):
                ctx[:, pl.ds((STRIDE * a + r) * BLK, BLK), :] = (
                    ctxr[:, a * BLK:(a + 1) * BLK, :]
                )

        for b in range(B):
            o_ref[b] = jnp.dot(
                ctx[b], wo_ref[...], preferred_element_type=jnp.float32
            )

    f32 = jnp.float32
    return pl.pallas_call(
        body,
        out_shape=jax.ShapeDtypeStruct((B, Sq, E), f32),
        in_specs=[pl.BlockSpec(memory_space=pltpu.VMEM)] * 5,
        out_specs=pl.BlockSpec(memory_space=pltpu.VMEM),
        scratch_shapes=[
            pltpu.VMEM((N_DEV, B, S, Fh), f32),
            pltpu.VMEM((N_DEV, B, S, Fh), f32),
            pltpu.VMEM((N_DEV, B, S, Fh), f32),
            pltpu.VMEM((N_DEV, B, S, Fh), f32),
            pltpu.VMEM((STRIDE, B, R, F), f32),
            pltpu.VMEM((STRIDE, B, R, F), f32),
            pltpu.VMEM((STRIDE, B, H, R), f32),
            pltpu.VMEM((B, Sq, F), f32),
        ] + [pltpu.SemaphoreType.DMA((N_DEV - 1,))] * 8,
        compiler_params=pltpu.CompilerParams(
            collective_id=0, vmem_limit_bytes=100 * 1024 * 1024,
        ),
    )(x, Wq, K_ext.reshape(B, S, F), V_ext.reshape(B, S, F), Wo)


# device time: 157633 ns/iter; 3.9759x vs baseline; 1.2909x over previous
import jax
import jax.numpy as jnp
from jax import lax
from jax.experimental import pallas as pl
from jax.experimental.pallas import tpu as pltpu

N_DEV = 8
NP = 4
BLK = 64
STRIDE = 4


def kernel(x, Wq, K_ext, V_ext, Wo):
    B, Sq, E = x.shape
    _, S, H, D = K_ext.shape
    F = H * D
    Hh = H // 2
    Fh = Hh * D
    R = Sq // STRIDE
    NB = Sq // (STRIDE * BLK)

    def body(x_ref, wq_ref, k_ref, v_ref, wo_ref, o_ref,
             klo, vlo, khi, vhi, qbuf, acc, lsum, ctx,
             cw_s, cw_r, ccw_s, ccw_r, z_s, z_r):
        my = lax.axis_index("i")
        pp = lax.rem(my, NP)
        pl4 = my - pp
        opl4 = NP - pl4
        cwn = pl4 + lax.rem(pp + 1, NP)
        ccwn = pl4 + lax.rem(pp + 3, NP)
        ptn = opl4 + pp

        bufs = (klo, vlo, khi, vhi)

        barrier = pltpu.get_barrier_semaphore()
        for nbr in (cwn, ccwn, ptn):
            pl.semaphore_signal(
                barrier, inc=1,
                device_id=(nbr,), device_id_type=pl.DeviceIdType.MESH,
            )
        pl.semaphore_wait(barrier, 3)

        klo[my] = k_ref[:, :, :Fh]
        vlo[my] = v_ref[:, :, :Fh]
        khi[my] = k_ref[:, :, Fh:]
        vhi[my] = v_ref[:, :, Fh:]

        acc[...] = jnp.zeros_like(acc)
        lsum[...] = jnp.zeros_like(lsum)

        def rdma(bi, slot, sems, rsems, h, tgt):
            return pltpu.make_async_remote_copy(
                src_ref=bufs[bi].at[slot], dst_ref=bufs[bi].at[slot],
                send_sem=sems.at[bi, h], recv_sem=rsems.at[bi, h],
                device_id=(tgt,), device_id_type=pl.DeviceIdType.MESH,
            )

        def plane_hop(h, cw_slot, cw_bis, ccw_slot, ccw_bis):
            return (
                [rdma(bi, cw_slot, cw_s, cw_r, h, cwn) for bi in cw_bis]
                + [rdma(bi, ccw_slot, ccw_s, ccw_r, h, ccwn) for bi in ccw_bis]
            )

        def start(rs):
            for r_ in rs:
                r_.start()

        def wait(rs):
            for r_ in rs:
                r_.wait()

        def process(kbuf, vbuf, slot, half):
            def rb_body(i, _):
                r = i // B
                b = lax.rem(i, B)
                kc = jnp.concatenate(
                    [kbuf[slot, b, pl.ds((r + STRIDE * a) * BLK, BLK), :]
                     for a in range(NB)], axis=0)
                vc = jnp.concatenate(
                    [vbuf[slot, b, pl.ds((r + STRIDE * a) * BLK, BLK), :]
                     for a in range(NB)], axis=0)
                q = qbuf[r, b, :, pl.ds(half * Fh, Fh)]
                s = jnp.einsum(
                    "ihd,jhd->hij",
                    q.reshape(R, Hh, D),
                    kc.reshape(NB * BLK, Hh, D),
                    preferred_element_type=jnp.float32,
                ) * 0.125
                e = jnp.exp(s)
                lsum[r, b, pl.ds(half * Hh, Hh), :] = (
                    lsum[r, b, pl.ds(half * Hh, Hh), :] + e.sum(axis=-1)
                )
                c = jnp.einsum(
                    "hij,jhd->ihd", e, vc.reshape(NB * BLK, Hh, D),
                    preferred_element_type=jnp.float32,
                ).reshape(R, Fh)
                acc[r, b, :, pl.ds(half * Fh, Fh)] = (
                    acc[r, b, :, pl.ds(half * Fh, Fh)] + c
                )
                return 0

            lax.fori_loop(0, STRIDE * B, rb_body, 0)

        LO, HI = (0, 1), (2, 3)

        z0 = [rdma(bi, my, z_s, z_r, 0, ptn) for bi in range(4)]
        h0 = plane_hop(0, my, LO, my, HI)
        start(z0)
        start(h0)

        for r in range(STRIDE):
            for b in range(B):
                xs = jnp.concatenate(
                    [x_ref[b, (r + STRIDE * a) * BLK:
                           (r + STRIDE * a + 1) * BLK, :] for a in range(NB)],
                    axis=0,
                )
                qbuf[r, b] = jnp.dot(
                    xs, wq_ref[...], preferred_element_type=jnp.float32
                )

        process(klo, vlo, my, 0)
        process(khi, vhi, my, 1)
        wait(h0)

        z1 = ([rdma(bi, ccwn, z_s, z_r, 1, ptn) for bi in LO]
              + [rdma(bi, cwn, z_s, z_r, 1, ptn) for bi in HI])
        h1 = plane_hop(1, ccwn, LO, cwn, HI)
        start(z1)
        start(h1)
        process(klo, vlo, ccwn, 0)
        process(khi, vhi, cwn, 1)
        wait(h1)

        far = pl4 + lax.rem(pp + 2, NP)
        h2 = plane_hop(2, far, LO, far, HI)
        start(h2)
        process(klo, vlo, far, 0)
        process(khi, vhi, far, 1)
        wait(h2)
        wait(z0)

        hA = plane_hop(3, ptn, HI, ptn, LO)
        start(hA)
        process(klo, vlo, cwn, 0)
        process(khi, vhi, ccwn, 1)
        process(klo, vlo, ptn, 0)
        process(khi, vhi, ptn, 1)
        wait(hA)
        wait(z1)

        z_prev = opl4 + lax.rem(pp + 3, NP)
        z_next = opl4 + lax.rem(pp + 1, NP)
        hB = plane_hop(4, z_prev, LO, z_next, HI)
        start(hB)
        process(klo, vlo, z_prev, 0)
        process(khi, vhi, z_prev, 1)
        process(klo, vlo, z_next, 0)
        process(khi, vhi, z_next, 1)
        wait(hB)

        z_far = opl4 + lax.rem(pp + 2, NP)
        process(klo, vlo, z_far, 0)
        process(khi, vhi, z_far, 1)

        for r in range(STRIDE):
            a_r = acc[r]
            l_r = lsum[r]
            ctxr = a_r.reshape(B, R, H, D) / jnp.transpose(
                l_r, (0, 2, 1)
            )[:, :, :, None]
            ctxr = ctxr.reshape(B, R, F)
            for a in range(NB):
                ctx[:, pl.ds((STRIDE * a + r) * BLK, BLK), :] = (
                    ctxr[:, a * BLK:(a + 1) * BLK, :]
                )

        for b in range(B):
            o_ref[b] = jnp.dot(
                ctx[b], wo_ref[...], preferred_element_type=jnp.float32
            )

    f32 = jnp.float32
    return pl.pallas_call(
        body,
        out_shape=jax.ShapeDtypeStruct((B, Sq, E), f32),
        in_specs=[pl.BlockSpec(memory_space=pltpu.VMEM)] * 5,
        out_specs=pl.BlockSpec(memory_space=pltpu.VMEM),
        scratch_shapes=[
            pltpu.VMEM((N_DEV, B, S, Fh), f32),
            pltpu.VMEM((N_DEV, B, S, Fh), f32),
            pltpu.VMEM((N_DEV, B, S, Fh), f32),
            pltpu.VMEM((N_DEV, B, S, Fh), f32),
            pltpu.VMEM((STRIDE, B, R, F), f32),
            pltpu.VMEM((STRIDE, B, R, F), f32),
            pltpu.VMEM((STRIDE, B, H, R), f32),
            pltpu.VMEM((B, Sq, F), f32),
            pltpu.SemaphoreType.DMA((4, 5)),
            pltpu.SemaphoreType.DMA((4, 5)),
            pltpu.SemaphoreType.DMA((4, 5)),
            pltpu.SemaphoreType.DMA((4, 5)),
            pltpu.SemaphoreType.DMA((4, 2)),
            pltpu.SemaphoreType.DMA((4, 2)),
        ],
        compiler_params=pltpu.CompilerParams(
            collective_id=0, vmem_limit_bytes=100 * 1024 * 1024,
        ),
    )(x, Wq, K_ext.reshape(B, S, F), V_ext.reshape(B, S, F), Wo)


# device time: 118677 ns/iter; 5.2810x vs baseline; 1.3283x over previous
import jax
import jax.numpy as jnp
from jax import lax
from jax.experimental import pallas as pl
from jax.experimental.pallas import tpu as pltpu

N_DEV = 8
NP = 4
BLK = 64
STRIDE = 4


def kernel(x, Wq, K_ext, V_ext, Wo):
    B, Sq, E = x.shape
    _, S, H, D = K_ext.shape
    F = H * D
    Hh = H // 2
    Fh = Hh * D
    R = Sq // STRIDE
    NB = Sq // (STRIDE * BLK)

    def body(x_ref, wq_ref, k_ref, v_ref, wo_ref, o_ref,
             klo, vlo, khi, vhi, qbuf, acc, lsum, ctx,
             cw_s, cw_r, ccw_s, ccw_r, z_s, z_r):
        my = lax.axis_index("i")
        pp = lax.rem(my, NP)
        pl4 = my - pp
        opl4 = NP - pl4
        cwn = pl4 + lax.rem(pp + 1, NP)
        ccwn = pl4 + lax.rem(pp + 3, NP)
        ptn = opl4 + pp

        bufs = (klo, vlo, khi, vhi)

        barrier = pltpu.get_barrier_semaphore()
        for nbr in (cwn, ccwn, ptn):
            pl.semaphore_signal(
                barrier, inc=1,
                device_id=(nbr,), device_id_type=pl.DeviceIdType.MESH,
            )
        pl.semaphore_wait(barrier, 3)

        klo[my] = k_ref[:, :, :Fh]
        vlo[my] = v_ref[:, :, :Fh]
        khi[my] = k_ref[:, :, Fh:]
        vhi[my] = v_ref[:, :, Fh:]

        acc[...] = jnp.zeros_like(acc)
        lsum[...] = jnp.zeros_like(lsum)

        def rdma(bi, slot, sems, rsems, h, tgt):
            return pltpu.make_async_remote_copy(
                src_ref=bufs[bi].at[slot], dst_ref=bufs[bi].at[slot],
                send_sem=sems.at[bi, h], recv_sem=rsems.at[bi, h],
                device_id=(tgt,), device_id_type=pl.DeviceIdType.MESH,
            )

        def plane_hop(h, cw_slot, cw_bis, ccw_slot, ccw_bis):
            return (
                [rdma(bi, cw_slot, cw_s, cw_r, h, cwn) for bi in cw_bis]
                + [rdma(bi, ccw_slot, ccw_s, ccw_r, h, ccwn) for bi in ccw_bis]
            )

        def start(rs):
            for r_ in rs:
                r_.start()

        def wait(rs):
            for r_ in rs:
                r_.wait()

        def process(kbuf, vbuf, slot, half):
            def rb_body(i, _):
                r = i // B
                b = lax.rem(i, B)
                kc = jnp.concatenate(
                    [kbuf[slot, b, pl.ds((r + STRIDE * a) * BLK, BLK), :]
                     for a in range(NB)], axis=0)
                vc = jnp.concatenate(
                    [vbuf[slot, b, pl.ds((r + STRIDE * a) * BLK, BLK), :]
                     for a in range(NB)], axis=0)
                q = qbuf[r, b, :, pl.ds(half * Fh, Fh)]
                s = jnp.einsum(
                    "ihd,jhd->hij",
                    q.reshape(R, Hh, D),
                    kc.reshape(NB * BLK, Hh, D),
                    preferred_element_type=jnp.float32,
                ) * 0.125
                e = jnp.exp(s)
                lsum[r, b, pl.ds(half * Hh, Hh), :] = (
                    lsum[r, b, pl.ds(half * Hh, Hh), :] + e.sum(axis=-1)
                )
                c = jnp.einsum(
                    "hij,jhd->ihd",
                    e.astype(jnp.bfloat16),
                    vc.reshape(NB * BLK, Hh, D),
                    preferred_element_type=jnp.float32,
                ).reshape(R, Fh)
                acc[r, b, :, pl.ds(half * Fh, Fh)] = (
                    acc[r, b, :, pl.ds(half * Fh, Fh)] + c
                )
                return 0

            lax.fori_loop(0, STRIDE * B, rb_body, 0)

        LO, HI = (0, 1), (2, 3)

        z0 = [rdma(bi, my, z_s, z_r, 0, ptn) for bi in range(4)]
        h0 = plane_hop(0, my, LO, my, HI)
        start(z0)
        start(h0)

        for r in range(STRIDE):
            for b in range(B):
                xs = jnp.concatenate(
                    [x_ref[b, (r + STRIDE * a) * BLK:
                           (r + STRIDE * a + 1) * BLK, :] for a in range(NB)],
                    axis=0,
                )
                qbuf[r, b] = jnp.dot(
                    xs, wq_ref[...], preferred_element_type=jnp.float32
                ).astype(jnp.bfloat16)

        process(klo, vlo, my, 0)
        process(khi, vhi, my, 1)
        wait(h0)

        z1 = ([rdma(bi, ccwn, z_s, z_r, 1, ptn) for bi in LO]
              + [rdma(bi, cwn, z_s, z_r, 1, ptn) for bi in HI])
        h1 = plane_hop(1, ccwn, LO, cwn, HI)
        start(z1)
        start(h1)
        process(klo, vlo, ccwn, 0)
        process(khi, vhi, cwn, 1)
        wait(h1)

        far = pl4 + lax.rem(pp + 2, NP)
        h2 = plane_hop(2, far, LO, far, HI)
        start(h2)
        process(klo, vlo, far, 0)
        process(khi, vhi, far, 1)
        wait(h2)
        wait(z0)

        hA = plane_hop(3, ptn, HI, ptn, LO)
        start(hA)
        process(klo, vlo, cwn, 0)
        process(khi, vhi, ccwn, 1)
        process(klo, vlo, ptn, 0)
        process(khi, vhi, ptn, 1)
        wait(hA)
        wait(z1)

        z_prev = opl4 + lax.rem(pp + 3, NP)
        z_next = opl4 + lax.rem(pp + 1, NP)
        hB = plane_hop(4, z_prev, LO, z_next, HI)
        start(hB)
        process(klo, vlo, z_prev, 0)
        process(khi, vhi, z_prev, 1)
        process(klo, vlo, z_next, 0)
        process(khi, vhi, z_next, 1)
        wait(hB)

        z_far = opl4 + lax.rem(pp + 2, NP)
        process(klo, vlo, z_far, 0)
        process(khi, vhi, z_far, 1)

        for r in range(STRIDE):
            a_r = acc[r]
            l_r = lsum[r]
            ctxr = a_r.reshape(B, R, H, D) / jnp.transpose(
                l_r, (0, 2, 1)
            )[:, :, :, None]
            ctxr = ctxr.reshape(B, R, F)
            for a in range(NB):
                ctx[:, pl.ds((STRIDE * a + r) * BLK, BLK), :] = (
                    ctxr[:, a * BLK:(a + 1) * BLK, :]
                )

        for b in range(B):
            o_ref[b] = jnp.dot(
                ctx[b], wo_ref[...], preferred_element_type=jnp.float32
            )

    f32 = jnp.float32
    bf16 = jnp.bfloat16
    return pl.pallas_call(
        body,
        out_shape=jax.ShapeDtypeStruct((B, Sq, E), f32),
        in_specs=[pl.BlockSpec(memory_space=pltpu.VMEM)] * 5,
        out_specs=pl.BlockSpec(memory_space=pltpu.VMEM),
        scratch_shapes=[
            pltpu.VMEM((N_DEV, B, S, Fh), bf16),
            pltpu.VMEM((N_DEV, B, S, Fh), bf16),
            pltpu.VMEM((N_DEV, B, S, Fh), bf16),
            pltpu.VMEM((N_DEV, B, S, Fh), bf16),
            pltpu.VMEM((STRIDE, B, R, F), bf16),
            pltpu.VMEM((STRIDE, B, R, F), f32),
            pltpu.VMEM((STRIDE, B, H, R), f32),
            pltpu.VMEM((B, Sq, F), f32),
            pltpu.SemaphoreType.DMA((4, 5)),
            pltpu.SemaphoreType.DMA((4, 5)),
            pltpu.SemaphoreType.DMA((4, 5)),
            pltpu.SemaphoreType.DMA((4, 5)),
            pltpu.SemaphoreType.DMA((4, 2)),
            pltpu.SemaphoreType.DMA((4, 2)),
        ],
        compiler_params=pltpu.CompilerParams(
            collective_id=0, vmem_limit_bytes=100 * 1024 * 1024,
        ),
    )(x, Wq,
      K_ext.reshape(B, S, F).astype(bf16),
      V_ext.reshape(B, S, F).astype(bf16), Wo)
